# baseline (speedup 1.0000x reference)
import jax
import jax.numpy as jnp
from jax import lax
from jax.experimental import pallas as pl
from jax.experimental.pallas import tpu as pltpu
from jax.experimental.pallas import tpu_sc as plsc

B, C, H, W, K = 16, 2, 512, 512, 256
HW = H * W
L = 16
NC = 2
NS = 16
NWORK = NC * NS
PAIRS = B * K
PW = PAIRS // NWORK
EW = PW * C
NB = 4
BCAP = EW + L
RCAP = EW + NB * 8 + L
DMA_ROWS = 8


def _sc_body(table, ind_flat, mask_flat, target_flat, part_hbm,
             ind_v, mask_v, tgt_v, row_v, col_v,
             bidx0, bidx1, bidx2, bidx3, beid0, beid1, beid2, beid3,
             rows_v, accs_v, red_v, shared, sem):
    bidx = (bidx0, bidx1, bidx2, bidx3)
    beid = (beid0, beid1, beid2, beid3)
    cid = lax.axis_index("c")
    sid = lax.axis_index("s")
    wid = cid * NS + sid
    base_pair = wid * PW
    pltpu.sync_copy(ind_flat.at[pl.ds(base_pair, PW)], ind_v)
    pltpu.sync_copy(mask_flat.at[pl.ds(base_pair, PW)], mask_v)
    pltpu.sync_copy(target_flat.at[pl.ds(base_pair * C, EW)], tgt_v)
    b = base_pair // K
    zeros16 = jnp.zeros((L,), jnp.int32)
    for k in range(NB):
        for i in range(BCAP // L):
            bidx[k][pl.ds(i * L, L)] = zeros16
            beid[k][pl.ds(i * L, L)] = zeros16
    for i in range(EW // L):
        e = lax.iota(jnp.int32, L) + i * L
        pair = e >> 1
        ch = e & 1
        p = plsc.load_gather(ind_v, [pair])
        row_v[pl.ds(i * L, L)] = (b * C + ch) * H + (p >> 9)
        col_v[pl.ds(i * L, L)] = p & (W - 1)
    cnt = [jnp.int32(0)] * NB
    for i in range(EW // L):
        e = lax.iota(jnp.int32, L) + i * L
        w = col_v[pl.ds(i * L, L)]
        r = row_v[pl.ds(i * L, L)]
        wt = w >> 7
        for k in range(NB):
            mk = wt == k
            ones = jnp.where(mk, jnp.int32(1), jnp.int32(0))
            pos = cnt[k] + plsc.cumsum(ones) - 1
            plsc.store_scatter(bidx[k], [pos], r, mask=mk)
            plsc.store_scatter(beid[k], [pos], e, mask=mk)
            cnt[k] = cnt[k] + jnp.sum(ones)
    base = [jnp.int32(0)] * NB
    for k in range(1, NB):
        base[k] = base[k - 1] + ((cnt[k - 1] + 7) & ~7)
    total_trips = jnp.int32(0)
    for k in range(NB):
        trips = (cnt[k] + (DMA_ROWS - 1)) // DMA_ROWS
        total_trips = total_trips + trips

        def fire(j, carry, k=k):
            pltpu.async_copy(
                table.at[bidx[k].at[pl.ds(j * DMA_ROWS, DMA_ROWS)],
                         pl.ds(k * 128, 128)],
                rows_v.at[pl.ds(base[k] + j * DMA_ROWS, DMA_ROWS)], sem)
            return carry
        lax.fori_loop(0, trips, fire, jnp.int32(0))

    drain = pltpu.make_async_copy(
        table.at[pl.ds(0, DMA_ROWS), pl.ds(0, 128)],
        rows_v.at[pl.ds(0, DMA_ROWS)], sem)

    def drain_one(j, carry):
        drain.wait()
        return carry
    lax.fori_loop(0, total_trips, drain_one, jnp.int32(0))
    acc = jnp.zeros((L,), jnp.float32)
    for k in range(NB):
        trips = (cnt[k] + (L - 1)) // L

        def body(j, a, k=k):
            lane = lax.iota(jnp.int32, L)
            s = j * L + lane
            valid = s < cnt[k]
            e = plsc.load_gather(beid[k], [s])
            w = plsc.load_gather(col_v, [e])
            vals = plsc.load_gather(rows_v, [base[k] + s, w & 127])
            m = plsc.load_gather(mask_v, [e >> 1])
            t = plsc.load_gather(tgt_v, [e])
            term = jnp.abs(vals * m - t * m)
            return a + jnp.where(valid, term, jnp.float32(0.0))
        acc = lax.fori_loop(0, trips, body, acc)
    macc = jnp.zeros((L,), jnp.float32)
    for i in range(EW // L):
        e = lax.iota(jnp.int32, L) + i * L
        macc = macc + plsc.load_gather(mask_v, [e >> 1])
    accs_v[pl.ds(0, L)] = acc
    accs_v[pl.ds(L, L)] = macc
    pltpu.sync_copy(accs_v, shared.at[pl.ds(sid * 2 * L, 2 * L)])

    plsc.subcore_barrier()

    @pl.when(sid == 0)
    def _reduce_core():
        pltpu.sync_copy(shared, red_v)
        a = jnp.zeros((L,), jnp.float32)
        ma = jnp.zeros((L,), jnp.float32)
        for w_ in range(NS):
            a = a + red_v[pl.ds(w_ * 2 * L, L)]
            ma = ma + red_v[pl.ds(w_ * 2 * L + L, L)]
        accs_v[pl.ds(0, L)] = a
        accs_v[pl.ds(L, L)] = ma
        pltpu.sync_copy(accs_v, part_hbm.at[cid])


_sc_launch = pl.kernel(
    _sc_body,
    out_type=jax.ShapeDtypeStruct((NC, 2 * L), jnp.float32),
    mesh=plsc.VectorSubcoreMesh(core_axis_name="c", subcore_axis_name="s"),
    compiler_params=pltpu.CompilerParams(
        needs_layout_passes=False, use_tc_tiling_on_sc=True),
    scratch_types=[
        pltpu.VMEM((PW,), jnp.int32),
        pltpu.VMEM((PW,), jnp.float32),
        pltpu.VMEM((EW,), jnp.float32),
        pltpu.VMEM((EW,), jnp.int32),
        pltpu.VMEM((EW,), jnp.int32),
        pltpu.VMEM((BCAP,), jnp.int32),
        pltpu.VMEM((BCAP,), jnp.int32),
        pltpu.VMEM((BCAP,), jnp.int32),
        pltpu.VMEM((BCAP,), jnp.int32),
        pltpu.VMEM((BCAP,), jnp.int32),
        pltpu.VMEM((BCAP,), jnp.int32),
        pltpu.VMEM((BCAP,), jnp.int32),
        pltpu.VMEM((BCAP,), jnp.int32),
        pltpu.VMEM((RCAP, 128), jnp.float32),
        pltpu.VMEM((2 * L,), jnp.float32),
        pltpu.VMEM((NS * 2 * L,), jnp.float32),
        pltpu.VMEM_SHARED((NS * 2 * L,), jnp.float32),
        pltpu.SemaphoreType.DMA,
    ],
)


def _combine_body(part_ref, out_ref):
    lsum = jnp.sum(part_ref[:, 0:L])
    msum = jnp.sum(part_ref[:, L:2 * L])
    out_ref[...] = jnp.full((1, 1), lsum / (msum + jnp.float32(1e-4)),
                            jnp.float32)


_combine = pl.pallas_call(
    _combine_body,
    out_shape=jax.ShapeDtypeStruct((1, 1), jnp.float32),
)


def kernel(output, mask, ind, target):
    table = output.reshape(B * C * H, W)
    ind_flat = ind.reshape(PAIRS)
    mask_flat = mask.reshape(PAIRS)
    target_flat = target.reshape(PAIRS * C)
    part = _sc_launch(table, ind_flat, mask_flat, target_flat)
    return _combine(part)[0, 0]


# speedup vs baseline: 1.0243x; 1.0243x over previous
import jax
import jax.numpy as jnp
from jax import lax
from jax.experimental import pallas as pl
from jax.experimental.pallas import tpu as pltpu
from jax.experimental.pallas import tpu_sc as plsc

B, C, H, W, K = 16, 2, 512, 512, 256
HW = H * W
L = 16
NC = 2
NS = 16
NWORK = NC * NS
PAIRS = B * K
PW = PAIRS // NWORK
EW = PW * C
ROWS_PER_DMA = 64
NPASS = EW // ROWS_PER_DMA
CHUNKS_PER_PASS = ROWS_PER_DMA // L


def _sc_body(table, ind_flat, mask_flat, target_flat, part_hbm,
             ind_v, mask_v, tgt_v, idx_v, col_v, rows0_v, rows1_v, term_v,
             accs_v, red_v, shared, sem0, sem1):
    cid = lax.axis_index("c")
    sid = lax.axis_index("s")
    wid = cid * NS + sid
    base_pair = wid * PW
    pltpu.sync_copy(ind_flat.at[pl.ds(base_pair, PW)], ind_v)
    pltpu.sync_copy(mask_flat.at[pl.ds(base_pair, PW)], mask_v)
    pltpu.sync_copy(target_flat.at[pl.ds(base_pair * C, EW)], tgt_v)
    b = base_pair // K
    for i in range(EW // L):
        e = lax.iota(jnp.int32, L) + i * L
        pair = e >> 1
        ch = e & 1
        p = plsc.load_gather(ind_v, [pair])
        idx_v[pl.ds(i * L, L)] = (b * C + ch) * H + (p >> 9)
        col_v[pl.ds(i * L, L)] = p & (W - 1)

    rows_bufs = (rows0_v, rows1_v)
    sems = (sem0, sem1)

    def fire(ps):
        return pltpu.async_copy(
            table.at[idx_v.at[pl.ds(ps * ROWS_PER_DMA, ROWS_PER_DMA)]],
            rows_bufs[ps % 2], sems[ps % 2])

    pending = fire(0)
    for ps in range(NPASS):
        pending.wait()
        if ps + 1 < NPASS:
            pending = fire(ps + 1)
        rows_v = rows_bufs[ps % 2]
        for i in range(CHUNKS_PER_PASS):
            off = ps * ROWS_PER_DMA + i * L
            pair = (lax.iota(jnp.int32, L) + off) >> 1
            e_loc = lax.iota(jnp.int32, L) + i * L
            col = col_v[pl.ds(off, L)]
            vals = plsc.load_gather(rows_v, [e_loc, col])
            m = plsc.load_gather(mask_v, [pair])
            t = tgt_v[pl.ds(off, L)]
            term_v[pl.ds(off, L)] = jnp.abs(vals * m - t * m)
    acc = jnp.zeros((L,), jnp.float32)
    macc = jnp.zeros((L,), jnp.float32)
    for i in range(EW // L):
        e = lax.iota(jnp.int32, L) + i * L
        acc = acc + term_v[pl.ds(i * L, L)]
        macc = macc + plsc.load_gather(mask_v, [e >> 1])
    accs_v[pl.ds(0, L)] = acc
    accs_v[pl.ds(L, L)] = macc
    pltpu.sync_copy(accs_v, shared.at[pl.ds(sid * 2 * L, 2 * L)])

    plsc.subcore_barrier()

    @pl.when(sid == 0)
    def _reduce_core():
        pltpu.sync_copy(shared, red_v)
        a = jnp.zeros((L,), jnp.float32)
        ma = jnp.zeros((L,), jnp.float32)
        for w_ in range(NS):
            a = a + red_v[pl.ds(w_ * 2 * L, L)]
            ma = ma + red_v[pl.ds(w_ * 2 * L + L, L)]
        accs_v[pl.ds(0, L)] = a
        accs_v[pl.ds(L, L)] = ma
        pltpu.sync_copy(accs_v, part_hbm.at[cid])


_sc_launch = pl.kernel(
    _sc_body,
    out_type=jax.ShapeDtypeStruct((NC, 2 * L), jnp.float32),
    mesh=plsc.VectorSubcoreMesh(core_axis_name="c", subcore_axis_name="s"),
    compiler_params=pltpu.CompilerParams(
        needs_layout_passes=False, use_tc_tiling_on_sc=True),
    scratch_types=[
        pltpu.VMEM((PW,), jnp.int32),
        pltpu.VMEM((PW,), jnp.float32),
        pltpu.VMEM((EW,), jnp.float32),
        pltpu.VMEM((EW,), jnp.int32),
        pltpu.VMEM((EW,), jnp.int32),
        pltpu.VMEM((ROWS_PER_DMA, W), jnp.float32),
        pltpu.VMEM((ROWS_PER_DMA, W), jnp.float32),
        pltpu.VMEM((EW,), jnp.float32),
        pltpu.VMEM((2 * L,), jnp.float32),
        pltpu.VMEM((NS * 2 * L,), jnp.float32),
        pltpu.VMEM_SHARED((NS * 2 * L,), jnp.float32),
        pltpu.SemaphoreType.DMA,
        pltpu.SemaphoreType.DMA,
    ],
)


def _combine_body(part_ref, out_ref):
    lsum = jnp.sum(part_ref[:, 0:L])
    msum = jnp.sum(part_ref[:, L:2 * L])
    out_ref[...] = jnp.full((1, 1), lsum / (msum + jnp.float32(1e-4)),
                            jnp.float32)


_combine = pl.pallas_call(
    _combine_body,
    out_shape=jax.ShapeDtypeStruct((1, 1), jnp.float32),
)


def kernel(output, mask, ind, target):
    table = output.reshape(B * C * H, W)
    ind_flat = ind.reshape(PAIRS)
    mask_flat = mask.reshape(PAIRS)
    target_flat = target.reshape(PAIRS * C)
    part = _sc_launch(table, ind_flat, mask_flat, target_flat)
    return _combine(part)[0, 0]
